# baseline (speedup 1.0000x reference)
import functools

import jax
import jax.numpy as jnp
from jax import lax
from jax.experimental import pallas as pl
from jax.experimental.pallas import tpu as pltpu
from jax.experimental.pallas import tpu_sc as plsc

_NUM_ANSWERS = 16384
_D = 16
_PACK = 8


@functools.lru_cache(maxsize=None)
def _build_sc_kernel():
    info = plsc.get_sparse_core_info()
    nc, ns = info.num_cores, info.num_subcores
    nw = nc * ns
    bpw = _NUM_ANSWERS // nw
    chunk = 256
    nchunks = bpw // chunk
    mesh = plsc.VectorSubcoreMesh(core_axis_name="c", subcore_axis_name="s")

    @functools.partial(
        pl.kernel,
        mesh=mesh,
        compiler_params=pltpu.CompilerParams(needs_layout_passes=False),
        out_type=(
            jax.ShapeDtypeStruct((_NUM_ANSWERS,), jnp.float32),
            jax.ShapeDtypeStruct((_NUM_ANSWERS,), jnp.float32),
        ),
        scratch_types=[
            pltpu.VMEM((bpw,), jnp.int32),
            pltpu.VMEM((bpw,), jnp.int32),
            pltpu.VMEM((bpw,), jnp.int32),
            pltpu.VMEM((bpw,), jnp.int32),
            pltpu.VMEM((bpw,), jnp.int32),
            pltpu.VMEM((bpw,), jnp.int32),
            pltpu.VMEM((chunk, _PACK * _D), jnp.float32),
            pltpu.VMEM((chunk, _PACK * _D), jnp.float32),
            pltpu.VMEM((chunk, _PACK * _D), jnp.float32),
            pltpu.VMEM((bpw,), jnp.float32),
            pltpu.VMEM((bpw,), jnp.float32),
            pltpu.SemaphoreType.DMA,
        ],
    )
    def sc_kernel(hidx_hbm, widx_hbm, lidx_hbm, table_hbm, win_hbm, lose_hbm,
                  hslot_v, wslot_v, lslot_v, hsub_v, wsub_v, lsub_v,
                  hbuf, wbuf, lbuf, win_v, lose_v, sem):
        wid = lax.axis_index("s") * nc + lax.axis_index("c")
        base = wid * bpw
        pltpu.sync_copy(hidx_hbm.at[pl.ds(base, bpw)], hslot_v)
        pltpu.sync_copy(widx_hbm.at[pl.ds(base, bpw)], wslot_v)
        pltpu.sync_copy(lidx_hbm.at[pl.ds(base, bpw)], lslot_v)

        def split_body(g, carry):
            off = g * 16
            for raw_v, slot_v, sub_v in (
                (hslot_v, hslot_v, hsub_v),
                (wslot_v, wslot_v, wsub_v),
                (lslot_v, lslot_v, lsub_v),
            ):
                raw = raw_v[pl.ds(off, 16)]
                sub_v[pl.ds(off, 16)] = (raw & (_PACK - 1)) * _D
                slot_v[pl.ds(off, 16)] = lax.shift_right_logical(raw, 3)
            return carry

        lax.fori_loop(0, bpw // 16, split_body, 0)

        last_lane = lax.iota(jnp.int32, 16) == 15

        def chunk_body(ci, carry):
            coff = ci * chunk
            cps = [
                pltpu.async_copy(
                    table_hbm.at[slot_v.at[pl.ds(coff, chunk)]], buf, sem)
                for slot_v, buf in
                ((hslot_v, hbuf), (wslot_v, wbuf), (lslot_v, lbuf))
            ]
            for cp in cps:
                cp.wait()

            def group_body(g, inner):
                goff = g * 16
                hs = hsub_v[pl.ds(coff + goff, 16)]
                ws = wsub_v[pl.ds(coff + goff, 16)]
                ls = lsub_v[pl.ds(coff + goff, 16)]
                for j in range(16):
                    i = goff + j
                    h = hbuf[i, pl.ds(hs[j], 16)]
                    w = wbuf[i, pl.ds(ws[j], 16)]
                    l = lbuf[i, pl.ds(ls[j], 16)]
                    dw = h - w
                    dl = h - l
                    iv = jnp.full((16,), coff + i, jnp.int32)
                    plsc.store_scatter(
                        win_v, [iv], plsc.cumsum(dw * dw), mask=last_lane)
                    plsc.store_scatter(
                        lose_v, [iv], plsc.cumsum(dl * dl), mask=last_lane)
                return inner

            lax.fori_loop(0, chunk // 16, group_body, 0)
            return carry

        lax.fori_loop(0, nchunks, chunk_body, 0)
        pltpu.sync_copy(win_v, win_hbm.at[pl.ds(base, bpw)])
        pltpu.sync_copy(lose_v, lose_hbm.at[pl.ds(base, bpw)])

    return sc_kernel


def kernel(h_w_l, embedding):
    rows_tbl = embedding.reshape(embedding.shape[0] // _PACK, _PACK * _D)
    hidx = h_w_l[:, 0]
    widx = h_w_l[:, 1]
    lidx = h_w_l[:, 2]
    win, lose = _build_sc_kernel()(hidx, widx, lidx, rows_tbl)
    return (win, lose)


# speedup vs baseline: 1.5586x; 1.5586x over previous
import functools

import jax
import jax.numpy as jnp
from jax import lax
from jax.experimental import pallas as pl
from jax.experimental.pallas import tpu as pltpu
from jax.experimental.pallas import tpu_sc as plsc

_NUM_ANSWERS = 16384
_N = 1000000
_D = 16
_PACK = 8
_TC_CHUNK = 8192
_TC_GRID = -(-_N // _TC_CHUNK)
_SLOTS = _TC_GRID * (_TC_CHUNK // _PACK)


def _tc_repack(table_t):
    def body(in_ref, out_ref):
        x = in_ref[...]
        sub = _TC_CHUNK // _PACK
        out_ref[...] = jnp.concatenate(
            [x[:, s * sub:(s + 1) * sub].T for s in range(_PACK)], axis=1)

    return pl.pallas_call(
        body,
        grid=(_TC_GRID,),
        in_specs=[pl.BlockSpec((_D, _TC_CHUNK), lambda j: (0, j))],
        out_specs=pl.BlockSpec((_TC_CHUNK // _PACK, _PACK * _D),
                               lambda j: (j, 0)),
        out_shape=jax.ShapeDtypeStruct((_SLOTS, _PACK * _D), jnp.float32),
    )(table_t)


@functools.lru_cache(maxsize=None)
def _build_sc_kernel():
    info = plsc.get_sparse_core_info()
    nc, ns = info.num_cores, info.num_subcores
    nw = nc * ns
    bpw = _NUM_ANSWERS // nw
    chunk = 256
    nchunks = bpw // chunk
    mesh = plsc.VectorSubcoreMesh(core_axis_name="c", subcore_axis_name="s")

    @functools.partial(
        pl.kernel,
        mesh=mesh,
        compiler_params=pltpu.CompilerParams(needs_layout_passes=False),
        out_type=(
            jax.ShapeDtypeStruct((_NUM_ANSWERS,), jnp.float32),
            jax.ShapeDtypeStruct((_NUM_ANSWERS,), jnp.float32),
        ),
        scratch_types=[
            pltpu.VMEM((bpw,), jnp.int32),
            pltpu.VMEM((bpw,), jnp.int32),
            pltpu.VMEM((bpw,), jnp.int32),
            pltpu.VMEM((bpw,), jnp.int32),
            pltpu.VMEM((bpw,), jnp.int32),
            pltpu.VMEM((bpw,), jnp.int32),
            pltpu.VMEM((chunk, _PACK * _D), jnp.float32),
            pltpu.VMEM((chunk, _PACK * _D), jnp.float32),
            pltpu.VMEM((chunk, _PACK * _D), jnp.float32),
            pltpu.VMEM((bpw,), jnp.float32),
            pltpu.VMEM((bpw,), jnp.float32),
            pltpu.SemaphoreType.DMA,
        ],
    )
    def sc_kernel(hidx_hbm, widx_hbm, lidx_hbm, table_hbm, win_hbm, lose_hbm,
                  hslot_v, wslot_v, lslot_v, hsub_v, wsub_v, lsub_v,
                  hbuf, wbuf, lbuf, win_v, lose_v, sem):
        wid = lax.axis_index("s") * nc + lax.axis_index("c")
        base = wid * bpw
        pltpu.sync_copy(hidx_hbm.at[pl.ds(base, bpw)], hslot_v)
        pltpu.sync_copy(widx_hbm.at[pl.ds(base, bpw)], wslot_v)
        pltpu.sync_copy(lidx_hbm.at[pl.ds(base, bpw)], lslot_v)

        def split_body(g, carry):
            off = g * 16
            for slot_v, sub_v in (
                (hslot_v, hsub_v), (wslot_v, wsub_v), (lslot_v, lsub_v),
            ):
                raw = slot_v[pl.ds(off, 16)]
                sub_v[pl.ds(off, 16)] = (
                    lax.shift_right_logical(raw, 10) & (_PACK - 1)) * _D
                slot_v[pl.ds(off, 16)] = (
                    lax.shift_left(lax.shift_right_logical(raw, 13), 10)
                    | (raw & 1023))
            return carry

        lax.fori_loop(0, bpw // 16, split_body, 0)

        last_lane = lax.iota(jnp.int32, 16) == 15

        def chunk_body(ci, carry):
            coff = ci * chunk
            cps = [
                pltpu.async_copy(
                    table_hbm.at[slot_v.at[pl.ds(coff, chunk)]], buf, sem)
                for slot_v, buf in
                ((hslot_v, hbuf), (wslot_v, wbuf), (lslot_v, lbuf))
            ]
            for cp in cps:
                cp.wait()

            def group_body(g, inner):
                goff = g * 16
                hs = hsub_v[pl.ds(coff + goff, 16)]
                ws = wsub_v[pl.ds(coff + goff, 16)]
                ls = lsub_v[pl.ds(coff + goff, 16)]
                for j in range(16):
                    i = goff + j
                    h = hbuf[i, pl.ds(hs[j], 16)]
                    w = wbuf[i, pl.ds(ws[j], 16)]
                    l = lbuf[i, pl.ds(ls[j], 16)]
                    dw = h - w
                    dl = h - l
                    iv = jnp.full((16,), coff + i, jnp.int32)
                    plsc.store_scatter(
                        win_v, [iv], plsc.cumsum(dw * dw), mask=last_lane)
                    plsc.store_scatter(
                        lose_v, [iv], plsc.cumsum(dl * dl), mask=last_lane)
                return inner

            lax.fori_loop(0, chunk // 16, group_body, 0)
            return carry

        lax.fori_loop(0, nchunks, chunk_body, 0)
        pltpu.sync_copy(win_v, win_hbm.at[pl.ds(base, bpw)])
        pltpu.sync_copy(lose_v, lose_hbm.at[pl.ds(base, bpw)])

    return sc_kernel


def kernel(h_w_l, embedding):
    table_t = embedding.T
    rows_tbl = _tc_repack(table_t)
    hidx = h_w_l[:, 0]
    widx = h_w_l[:, 1]
    lidx = h_w_l[:, 2]
    win, lose = _build_sc_kernel()(hidx, widx, lidx, rows_tbl)
    return (win, lose)


# speedup vs baseline: 4.3293x; 2.7776x over previous
import functools

import jax
import jax.numpy as jnp
from jax import lax
from jax.experimental import pallas as pl
from jax.experimental.pallas import tpu as pltpu
from jax.experimental.pallas import tpu_sc as plsc

_NUM_ANSWERS = 16384
_N = 1000000
_D = 16
_PACK = 8
_TC_CHUNK = 16384
_TC_GRID = -(-_N // _TC_CHUNK)
_SLOTS = _TC_GRID * (_TC_CHUNK // _PACK)
_SUB = _TC_CHUNK // _PACK


def _tc_repack(table_t):
    def body(in_ref, out_ref):
        x = in_ref[...]
        xs = jnp.concatenate(
            [x[:, s * _SUB:(s + 1) * _SUB] for s in range(_PACK)], axis=0)
        r_iota = lax.broadcasted_iota(jnp.int32, (_PACK * _D, _PACK * _D), 0)
        c_iota = lax.broadcasted_iota(jnp.int32, (_PACK * _D, _PACK * _D), 1)
        perm = jnp.where(
            c_iota == (r_iota & (_D - 1)) * _PACK
            + lax.shift_right_logical(r_iota, 4),
            1.0, 0.0)
        out_ref[...] = lax.dot_general(
            xs, perm, (((0,), (0,)), ((), ())),
            preferred_element_type=jnp.float32)

    return pl.pallas_call(
        body,
        grid=(_TC_GRID,),
        in_specs=[pl.BlockSpec((_D, _TC_CHUNK), lambda j: (0, j))],
        out_specs=pl.BlockSpec((_SUB, _PACK * _D), lambda j: (j, 0)),
        out_shape=jax.ShapeDtypeStruct((_SLOTS, _PACK * _D), jnp.float32),
    )(table_t)


@functools.lru_cache(maxsize=None)
def _build_sc_kernel():
    info = plsc.get_sparse_core_info()
    nc, ns = info.num_cores, info.num_subcores
    nw = nc * ns
    bpw = _NUM_ANSWERS // nw
    chunk = 256
    nchunks = bpw // chunk
    mesh = plsc.VectorSubcoreMesh(core_axis_name="c", subcore_axis_name="s")

    @functools.partial(
        pl.kernel,
        mesh=mesh,
        compiler_params=pltpu.CompilerParams(needs_layout_passes=False),
        out_type=(
            jax.ShapeDtypeStruct((_NUM_ANSWERS,), jnp.float32),
            jax.ShapeDtypeStruct((_NUM_ANSWERS,), jnp.float32),
        ),
        scratch_types=[
            pltpu.VMEM((bpw,), jnp.int32),
            pltpu.VMEM((bpw,), jnp.int32),
            pltpu.VMEM((bpw,), jnp.int32),
            pltpu.VMEM((bpw,), jnp.int32),
            pltpu.VMEM((bpw,), jnp.int32),
            pltpu.VMEM((bpw,), jnp.int32),
            pltpu.VMEM((chunk, _PACK * _D), jnp.float32),
            pltpu.VMEM((chunk, _PACK * _D), jnp.float32),
            pltpu.VMEM((chunk, _PACK * _D), jnp.float32),
            pltpu.VMEM((bpw,), jnp.float32),
            pltpu.VMEM((bpw,), jnp.float32),
            pltpu.SemaphoreType.DMA,
        ],
    )
    def sc_kernel(hidx_hbm, widx_hbm, lidx_hbm, table_hbm, win_hbm, lose_hbm,
                  hslot_v, wslot_v, lslot_v, hsub_v, wsub_v, lsub_v,
                  hbuf, wbuf, lbuf, win_v, lose_v, sem):
        wid = lax.axis_index("s") * nc + lax.axis_index("c")
        base = wid * bpw
        pltpu.sync_copy(hidx_hbm.at[pl.ds(base, bpw)], hslot_v)
        pltpu.sync_copy(widx_hbm.at[pl.ds(base, bpw)], wslot_v)
        pltpu.sync_copy(lidx_hbm.at[pl.ds(base, bpw)], lslot_v)

        def split_body(g, carry):
            off = g * 16
            for slot_v, sub_v in (
                (hslot_v, hsub_v), (wslot_v, wsub_v), (lslot_v, lsub_v),
            ):
                raw = slot_v[pl.ds(off, 16)]
                sub_v[pl.ds(off, 16)] = (
                    lax.shift_right_logical(raw, 11) & (_PACK - 1))
                slot_v[pl.ds(off, 16)] = (
                    lax.shift_left(lax.shift_right_logical(raw, 14), 11)
                    | (raw & 2047))
            return carry

        lax.fori_loop(0, bpw // 16, split_body, 0)

        last_lane = lax.iota(jnp.int32, 16) == 15
        lane_base = lax.iota(jnp.int32, 16) * _PACK

        def chunk_body(ci, carry):
            coff = ci * chunk
            cps = [
                pltpu.async_copy(
                    table_hbm.at[slot_v.at[pl.ds(coff, chunk)]], buf, sem)
                for slot_v, buf in
                ((hslot_v, hbuf), (wslot_v, wbuf), (lslot_v, lbuf))
            ]
            for cp in cps:
                cp.wait()

            def group_body(g, inner):
                goff = g * 16
                hs = hsub_v[pl.ds(coff + goff, 16)]
                ws = wsub_v[pl.ds(coff + goff, 16)]
                ls = lsub_v[pl.ds(coff + goff, 16)]
                for j in range(16):
                    i = goff + j
                    row = jnp.full((16,), i, jnp.int32)
                    h = plsc.load_gather(hbuf, [row, lane_base + hs[j]])
                    w = plsc.load_gather(wbuf, [row, lane_base + ws[j]])
                    l = plsc.load_gather(lbuf, [row, lane_base + ls[j]])
                    dw = h - w
                    dl = h - l
                    iv = jnp.full((16,), coff + i, jnp.int32)
                    plsc.store_scatter(
                        win_v, [iv], plsc.cumsum(dw * dw), mask=last_lane)
                    plsc.store_scatter(
                        lose_v, [iv], plsc.cumsum(dl * dl), mask=last_lane)
                return inner

            lax.fori_loop(0, chunk // 16, group_body, 0)
            return carry

        lax.fori_loop(0, nchunks, chunk_body, 0)
        pltpu.sync_copy(win_v, win_hbm.at[pl.ds(base, bpw)])
        pltpu.sync_copy(lose_v, lose_hbm.at[pl.ds(base, bpw)])

    return sc_kernel


def kernel(h_w_l, embedding):
    table_t = embedding.T
    rows_tbl = _tc_repack(table_t)
    hidx = h_w_l[:, 0]
    widx = h_w_l[:, 1]
    lidx = h_w_l[:, 2]
    win, lose = _build_sc_kernel()(hidx, widx, lidx, rows_tbl)
    return (win, lose)


# speedup vs baseline: 5.2714x; 1.2176x over previous
import functools

import jax
import jax.numpy as jnp
from jax import lax
from jax.experimental import pallas as pl
from jax.experimental.pallas import tpu as pltpu
from jax.experimental.pallas import tpu_sc as plsc

_NUM_ANSWERS = 16384
_N = 1000000
_D = 16
_PACK = 8
_TC_CHUNK = 32768
_TC_GRID = -(-_N // _TC_CHUNK)
_SLOTS = _TC_GRID * (_TC_CHUNK // _PACK)
_SUB = _TC_CHUNK // _PACK
_SUB_SHIFT = _SUB.bit_length() - 1
_CHUNK_SHIFT = _TC_CHUNK.bit_length() - 1


def _tc_repack(table_t):
    def body(in_ref, out_ref):
        x = in_ref[...]
        xs = jnp.concatenate(
            [x[:, s * _SUB:(s + 1) * _SUB] for s in range(_PACK)], axis=0)
        r_iota = lax.broadcasted_iota(jnp.int32, (_PACK * _D, _PACK * _D), 0)
        c_iota = lax.broadcasted_iota(jnp.int32, (_PACK * _D, _PACK * _D), 1)
        perm = jnp.where(
            c_iota == (r_iota & (_D - 1)) * _PACK
            + lax.shift_right_logical(r_iota, 4),
            1.0, 0.0)
        out_ref[...] = lax.dot_general(
            xs, perm, (((0,), (0,)), ((), ())),
            preferred_element_type=jnp.float32)

    return pl.pallas_call(
        body,
        grid=(_TC_GRID,),
        in_specs=[pl.BlockSpec((_D, _TC_CHUNK), lambda j: (0, j))],
        out_specs=pl.BlockSpec((_SUB, _PACK * _D), lambda j: (j, 0)),
        out_shape=jax.ShapeDtypeStruct((_SLOTS, _PACK * _D), jnp.float32),
    )(table_t)


@functools.lru_cache(maxsize=None)
def _build_sc_kernel():
    info = plsc.get_sparse_core_info()
    nc, ns = info.num_cores, info.num_subcores
    nw = nc * ns
    bpw = _NUM_ANSWERS // nw
    chunk = 256
    nchunks = bpw // chunk
    mesh = plsc.VectorSubcoreMesh(core_axis_name="c", subcore_axis_name="s")

    @functools.partial(
        pl.kernel,
        mesh=mesh,
        compiler_params=pltpu.CompilerParams(needs_layout_passes=False),
        out_type=(
            jax.ShapeDtypeStruct((_NUM_ANSWERS,), jnp.float32),
            jax.ShapeDtypeStruct((_NUM_ANSWERS,), jnp.float32),
        ),
        scratch_types=[
            pltpu.VMEM((bpw,), jnp.int32),
            pltpu.VMEM((bpw,), jnp.int32),
            pltpu.VMEM((bpw,), jnp.int32),
            pltpu.VMEM((bpw,), jnp.int32),
            pltpu.VMEM((bpw,), jnp.int32),
            pltpu.VMEM((bpw,), jnp.int32),
            pltpu.VMEM((chunk, _PACK * _D), jnp.float32),
            pltpu.VMEM((chunk, _PACK * _D), jnp.float32),
            pltpu.VMEM((chunk, _PACK * _D), jnp.float32),
            pltpu.VMEM((bpw,), jnp.float32),
            pltpu.VMEM((bpw,), jnp.float32),
            pltpu.SemaphoreType.DMA,
        ],
    )
    def sc_kernel(hidx_hbm, widx_hbm, lidx_hbm, table_hbm, win_hbm, lose_hbm,
                  hslot_v, wslot_v, lslot_v, hsub_v, wsub_v, lsub_v,
                  hbuf, wbuf, lbuf, win_v, lose_v, sem):
        wid = lax.axis_index("s") * nc + lax.axis_index("c")
        base = wid * bpw
        pltpu.sync_copy(hidx_hbm.at[pl.ds(base, bpw)], hslot_v)
        pltpu.sync_copy(widx_hbm.at[pl.ds(base, bpw)], wslot_v)
        pltpu.sync_copy(lidx_hbm.at[pl.ds(base, bpw)], lslot_v)

        def split_body(g, carry):
            off = g * 16
            for slot_v, sub_v in (
                (hslot_v, hsub_v), (wslot_v, wsub_v), (lslot_v, lsub_v),
            ):
                raw = slot_v[pl.ds(off, 16)]
                sub_v[pl.ds(off, 16)] = (
                    lax.shift_right_logical(raw, _SUB_SHIFT) & (_PACK - 1))
                slot_v[pl.ds(off, 16)] = (
                    lax.shift_left(
                        lax.shift_right_logical(raw, _CHUNK_SHIFT), _SUB_SHIFT)
                    | (raw & (_SUB - 1)))
            return carry

        lax.fori_loop(0, bpw // 16, split_body, 0)

        last_lane = lax.iota(jnp.int32, 16) == 15
        lane_base = lax.iota(jnp.int32, 16) * _PACK

        def chunk_body(ci, carry):
            coff = ci * chunk
            cps = [
                pltpu.async_copy(
                    table_hbm.at[slot_v.at[pl.ds(coff, chunk)]], buf, sem)
                for slot_v, buf in
                ((hslot_v, hbuf), (wslot_v, wbuf), (lslot_v, lbuf))
            ]
            for cp in cps:
                cp.wait()

            def group_body(g, inner):
                goff = g * 16
                hs = hsub_v[pl.ds(coff + goff, 16)]
                ws = wsub_v[pl.ds(coff + goff, 16)]
                ls = lsub_v[pl.ds(coff + goff, 16)]
                for j in range(16):
                    i = goff + j
                    row = jnp.full((16,), i, jnp.int32)
                    h = plsc.load_gather(hbuf, [row, lane_base + hs[j]])
                    w = plsc.load_gather(wbuf, [row, lane_base + ws[j]])
                    l = plsc.load_gather(lbuf, [row, lane_base + ls[j]])
                    dw = h - w
                    dl = h - l
                    iv = jnp.full((16,), coff + i, jnp.int32)
                    plsc.store_scatter(
                        win_v, [iv], plsc.cumsum(dw * dw), mask=last_lane)
                    plsc.store_scatter(
                        lose_v, [iv], plsc.cumsum(dl * dl), mask=last_lane)
                return inner

            lax.fori_loop(0, chunk // 16, group_body, 0)
            return carry

        lax.fori_loop(0, nchunks, chunk_body, 0)
        pltpu.sync_copy(win_v, win_hbm.at[pl.ds(base, bpw)])
        pltpu.sync_copy(lose_v, lose_hbm.at[pl.ds(base, bpw)])

    return sc_kernel


def kernel(h_w_l, embedding):
    table_t = embedding.T
    rows_tbl = _tc_repack(table_t)
    hidx = h_w_l[:, 0]
    widx = h_w_l[:, 1]
    lidx = h_w_l[:, 2]
    win, lose = _build_sc_kernel()(hidx, widx, lidx, rows_tbl)
    return (win, lose)


# speedup vs baseline: 5.7410x; 1.0891x over previous
import functools

import jax
import jax.numpy as jnp
from jax import lax
from jax.experimental import pallas as pl
from jax.experimental.pallas import tpu as pltpu
from jax.experimental.pallas import tpu_sc as plsc

_NUM_ANSWERS = 16384
_N = 1000000
_D = 16
_PACK = 8
_TC_CHUNK = 65536
_TC_GRID = -(-_N // _TC_CHUNK)
_SLOTS = _TC_GRID * (_TC_CHUNK // _PACK)
_SUB = _TC_CHUNK // _PACK
_SUB_SHIFT = _SUB.bit_length() - 1
_CHUNK_SHIFT = _TC_CHUNK.bit_length() - 1


def _tc_repack(table_t):
    def body(in_ref, out_ref):
        x = in_ref[...]
        xs = jnp.concatenate(
            [x[:, s * _SUB:(s + 1) * _SUB] for s in range(_PACK)], axis=0)
        r_iota = lax.broadcasted_iota(jnp.int32, (_PACK * _D, _PACK * _D), 0)
        c_iota = lax.broadcasted_iota(jnp.int32, (_PACK * _D, _PACK * _D), 1)
        perm = jnp.where(
            c_iota == (r_iota & (_D - 1)) * _PACK
            + lax.shift_right_logical(r_iota, 4),
            1.0, 0.0)
        out_ref[...] = lax.dot_general(
            xs, perm, (((0,), (0,)), ((), ())),
            preferred_element_type=jnp.float32)

    return pl.pallas_call(
        body,
        grid=(_TC_GRID,),
        in_specs=[pl.BlockSpec((_D, _TC_CHUNK), lambda j: (0, j))],
        out_specs=pl.BlockSpec((_SUB, _PACK * _D), lambda j: (j, 0)),
        out_shape=jax.ShapeDtypeStruct((_SLOTS, _PACK * _D), jnp.float32),
    )(table_t)


@functools.lru_cache(maxsize=None)
def _build_sc_kernel():
    info = plsc.get_sparse_core_info()
    nc, ns = info.num_cores, info.num_subcores
    nw = nc * ns
    bpw = _NUM_ANSWERS // nw
    chunk = 256
    nchunks = bpw // chunk
    mesh = plsc.VectorSubcoreMesh(core_axis_name="c", subcore_axis_name="s")

    @functools.partial(
        pl.kernel,
        mesh=mesh,
        compiler_params=pltpu.CompilerParams(needs_layout_passes=False),
        out_type=(
            jax.ShapeDtypeStruct((_NUM_ANSWERS,), jnp.float32),
            jax.ShapeDtypeStruct((_NUM_ANSWERS,), jnp.float32),
        ),
        scratch_types=[
            pltpu.VMEM((bpw,), jnp.int32),
            pltpu.VMEM((bpw,), jnp.int32),
            pltpu.VMEM((bpw,), jnp.int32),
            pltpu.VMEM((bpw,), jnp.int32),
            pltpu.VMEM((bpw,), jnp.int32),
            pltpu.VMEM((bpw,), jnp.int32),
            pltpu.VMEM((chunk, _PACK * _D), jnp.float32),
            pltpu.VMEM((chunk, _PACK * _D), jnp.float32),
            pltpu.VMEM((chunk, _PACK * _D), jnp.float32),
            pltpu.VMEM((bpw,), jnp.float32),
            pltpu.VMEM((bpw,), jnp.float32),
            pltpu.SemaphoreType.DMA,
        ],
    )
    def sc_kernel(hidx_hbm, widx_hbm, lidx_hbm, table_hbm, win_hbm, lose_hbm,
                  hslot_v, wslot_v, lslot_v, hsub_v, wsub_v, lsub_v,
                  hbuf, wbuf, lbuf, win_v, lose_v, sem):
        wid = lax.axis_index("s") * nc + lax.axis_index("c")
        base = wid * bpw
        pltpu.sync_copy(hidx_hbm.at[pl.ds(base, bpw)], hslot_v)
        pltpu.sync_copy(widx_hbm.at[pl.ds(base, bpw)], wslot_v)
        pltpu.sync_copy(lidx_hbm.at[pl.ds(base, bpw)], lslot_v)

        def split_body(g, carry):
            off = g * 16
            for slot_v, sub_v in (
                (hslot_v, hsub_v), (wslot_v, wsub_v), (lslot_v, lsub_v),
            ):
                raw = slot_v[pl.ds(off, 16)]
                sub_v[pl.ds(off, 16)] = (
                    lax.shift_right_logical(raw, _SUB_SHIFT) & (_PACK - 1))
                slot_v[pl.ds(off, 16)] = (
                    lax.shift_left(
                        lax.shift_right_logical(raw, _CHUNK_SHIFT), _SUB_SHIFT)
                    | (raw & (_SUB - 1)))
            return carry

        lax.fori_loop(0, bpw // 16, split_body, 0)

        last_lane = lax.iota(jnp.int32, 16) == 15
        lane_base = lax.iota(jnp.int32, 16) * _PACK

        def chunk_body(ci, carry):
            coff = ci * chunk
            cps = [
                pltpu.async_copy(
                    table_hbm.at[slot_v.at[pl.ds(coff, chunk)]], buf, sem)
                for slot_v, buf in
                ((hslot_v, hbuf), (wslot_v, wbuf), (lslot_v, lbuf))
            ]
            for cp in cps:
                cp.wait()

            def group_body(g, inner):
                goff = g * 16
                hs = hsub_v[pl.ds(coff + goff, 16)]
                ws = wsub_v[pl.ds(coff + goff, 16)]
                ls = lsub_v[pl.ds(coff + goff, 16)]
                for j in range(16):
                    i = goff + j
                    row = jnp.full((16,), i, jnp.int32)
                    h = plsc.load_gather(hbuf, [row, lane_base + hs[j]])
                    w = plsc.load_gather(wbuf, [row, lane_base + ws[j]])
                    l = plsc.load_gather(lbuf, [row, lane_base + ls[j]])
                    dw = h - w
                    dl = h - l
                    iv = jnp.full((16,), coff + i, jnp.int32)
                    plsc.store_scatter(
                        win_v, [iv], plsc.cumsum(dw * dw), mask=last_lane)
                    plsc.store_scatter(
                        lose_v, [iv], plsc.cumsum(dl * dl), mask=last_lane)
                return inner

            lax.fori_loop(0, chunk // 16, group_body, 0)
            return carry

        lax.fori_loop(0, nchunks, chunk_body, 0)
        pltpu.sync_copy(win_v, win_hbm.at[pl.ds(base, bpw)])
        pltpu.sync_copy(lose_v, lose_hbm.at[pl.ds(base, bpw)])

    return sc_kernel


def kernel(h_w_l, embedding):
    table_t = embedding.T
    rows_tbl = _tc_repack(table_t)
    hidx = h_w_l[:, 0]
    widx = h_w_l[:, 1]
    lidx = h_w_l[:, 2]
    win, lose = _build_sc_kernel()(hidx, widx, lidx, rows_tbl)
    return (win, lose)


# speedup vs baseline: 5.8201x; 1.0138x over previous
import functools

import jax
import jax.numpy as jnp
from jax import lax
from jax.experimental import pallas as pl
from jax.experimental.pallas import tpu as pltpu
from jax.experimental.pallas import tpu_sc as plsc

_NUM_ANSWERS = 16384
_N = 1000000
_D = 16
_PACK = 8
_TC_CHUNK = 131072
_TC_GRID = -(-_N // _TC_CHUNK)
_SLOTS = _TC_GRID * (_TC_CHUNK // _PACK)
_SUB = _TC_CHUNK // _PACK
_SUB_SHIFT = _SUB.bit_length() - 1
_CHUNK_SHIFT = _TC_CHUNK.bit_length() - 1


def _tc_repack(table_t):
    def body(in_ref, out_ref):
        x = in_ref[...]
        xs = jnp.concatenate(
            [x[:, s * _SUB:(s + 1) * _SUB] for s in range(_PACK)], axis=0)
        r_iota = lax.broadcasted_iota(jnp.int32, (_PACK * _D, _PACK * _D), 0)
        c_iota = lax.broadcasted_iota(jnp.int32, (_PACK * _D, _PACK * _D), 1)
        perm = jnp.where(
            c_iota == (r_iota & (_D - 1)) * _PACK
            + lax.shift_right_logical(r_iota, 4),
            1.0, 0.0)
        out_ref[...] = lax.dot_general(
            xs, perm, (((0,), (0,)), ((), ())),
            preferred_element_type=jnp.float32)

    return pl.pallas_call(
        body,
        grid=(_TC_GRID,),
        in_specs=[pl.BlockSpec((_D, _TC_CHUNK), lambda j: (0, j))],
        out_specs=pl.BlockSpec((_SUB, _PACK * _D), lambda j: (j, 0)),
        out_shape=jax.ShapeDtypeStruct((_SLOTS, _PACK * _D), jnp.float32),
    )(table_t)


@functools.lru_cache(maxsize=None)
def _build_sc_kernel():
    info = plsc.get_sparse_core_info()
    nc, ns = info.num_cores, info.num_subcores
    nw = nc * ns
    bpw = _NUM_ANSWERS // nw
    chunk = 256
    nchunks = bpw // chunk
    mesh = plsc.VectorSubcoreMesh(core_axis_name="c", subcore_axis_name="s")

    @functools.partial(
        pl.kernel,
        mesh=mesh,
        compiler_params=pltpu.CompilerParams(needs_layout_passes=False),
        out_type=(
            jax.ShapeDtypeStruct((_NUM_ANSWERS,), jnp.float32),
            jax.ShapeDtypeStruct((_NUM_ANSWERS,), jnp.float32),
        ),
        scratch_types=[
            pltpu.VMEM((bpw,), jnp.int32),
            pltpu.VMEM((bpw,), jnp.int32),
            pltpu.VMEM((bpw,), jnp.int32),
            pltpu.VMEM((bpw,), jnp.int32),
            pltpu.VMEM((bpw,), jnp.int32),
            pltpu.VMEM((bpw,), jnp.int32),
            pltpu.VMEM((chunk, _PACK * _D), jnp.float32),
            pltpu.VMEM((chunk, _PACK * _D), jnp.float32),
            pltpu.VMEM((chunk, _PACK * _D), jnp.float32),
            pltpu.VMEM((bpw,), jnp.float32),
            pltpu.VMEM((bpw,), jnp.float32),
            pltpu.SemaphoreType.DMA,
        ],
    )
    def sc_kernel(hidx_hbm, widx_hbm, lidx_hbm, table_hbm, win_hbm, lose_hbm,
                  hslot_v, wslot_v, lslot_v, hsub_v, wsub_v, lsub_v,
                  hbuf, wbuf, lbuf, win_v, lose_v, sem):
        wid = lax.axis_index("s") * nc + lax.axis_index("c")
        base = wid * bpw
        pltpu.sync_copy(hidx_hbm.at[pl.ds(base, bpw)], hslot_v)
        pltpu.sync_copy(widx_hbm.at[pl.ds(base, bpw)], wslot_v)
        pltpu.sync_copy(lidx_hbm.at[pl.ds(base, bpw)], lslot_v)

        def split_body(g, carry):
            off = g * 16
            for slot_v, sub_v in (
                (hslot_v, hsub_v), (wslot_v, wsub_v), (lslot_v, lsub_v),
            ):
                raw = slot_v[pl.ds(off, 16)]
                sub_v[pl.ds(off, 16)] = (
                    lax.shift_right_logical(raw, _SUB_SHIFT) & (_PACK - 1))
                slot_v[pl.ds(off, 16)] = (
                    lax.shift_left(
                        lax.shift_right_logical(raw, _CHUNK_SHIFT), _SUB_SHIFT)
                    | (raw & (_SUB - 1)))
            return carry

        lax.fori_loop(0, bpw // 16, split_body, 0)

        last_lane = lax.iota(jnp.int32, 16) == 15
        lane_base = lax.iota(jnp.int32, 16) * _PACK

        def chunk_body(ci, carry):
            coff = ci * chunk
            cps = [
                pltpu.async_copy(
                    table_hbm.at[slot_v.at[pl.ds(coff, chunk)]], buf, sem)
                for slot_v, buf in
                ((hslot_v, hbuf), (wslot_v, wbuf), (lslot_v, lbuf))
            ]
            for cp in cps:
                cp.wait()

            def group_body(g, inner):
                goff = g * 16
                hs = hsub_v[pl.ds(coff + goff, 16)]
                ws = wsub_v[pl.ds(coff + goff, 16)]
                ls = lsub_v[pl.ds(coff + goff, 16)]
                for j in range(16):
                    i = goff + j
                    row = jnp.full((16,), i, jnp.int32)
                    h = plsc.load_gather(hbuf, [row, lane_base + hs[j]])
                    w = plsc.load_gather(wbuf, [row, lane_base + ws[j]])
                    l = plsc.load_gather(lbuf, [row, lane_base + ls[j]])
                    dw = h - w
                    dl = h - l
                    iv = jnp.full((16,), coff + i, jnp.int32)
                    plsc.store_scatter(
                        win_v, [iv], plsc.cumsum(dw * dw), mask=last_lane)
                    plsc.store_scatter(
                        lose_v, [iv], plsc.cumsum(dl * dl), mask=last_lane)
                return inner

            lax.fori_loop(0, chunk // 16, group_body, 0)
            return carry

        lax.fori_loop(0, nchunks, chunk_body, 0)
        pltpu.sync_copy(win_v, win_hbm.at[pl.ds(base, bpw)])
        pltpu.sync_copy(lose_v, lose_hbm.at[pl.ds(base, bpw)])

    return sc_kernel


def kernel(h_w_l, embedding):
    table_t = embedding.T
    rows_tbl = _tc_repack(table_t)
    hidx = h_w_l[:, 0]
    widx = h_w_l[:, 1]
    lidx = h_w_l[:, 2]
    win, lose = _build_sc_kernel()(hidx, widx, lidx, rows_tbl)
    return (win, lose)


# speedup vs baseline: 6.1368x; 1.0544x over previous
import functools

import jax
import jax.numpy as jnp
from jax import lax
from jax.experimental import pallas as pl
from jax.experimental.pallas import tpu as pltpu
from jax.experimental.pallas import tpu_sc as plsc

_NUM_ANSWERS = 16384
_N = 1000000
_D = 16
_PACK = 8
_TC_CHUNK = 131072
_TC_GRID = -(-_N // _TC_CHUNK)
_SLOTS = _TC_GRID * (_TC_CHUNK // _PACK)
_SUB = _TC_CHUNK // _PACK
_SUB_SHIFT = _SUB.bit_length() - 1
_CHUNK_SHIFT = _TC_CHUNK.bit_length() - 1


def _tc_repack(table_t):
    def body(in_ref, out_ref):
        x = in_ref[...]
        xs = jnp.concatenate(
            [x[:, s * _SUB:(s + 1) * _SUB] for s in range(_PACK)], axis=0)
        r_iota = lax.broadcasted_iota(jnp.int32, (_PACK * _D, _PACK * _D), 0)
        c_iota = lax.broadcasted_iota(jnp.int32, (_PACK * _D, _PACK * _D), 1)
        perm = jnp.where(
            c_iota == (r_iota & (_D - 1)) * _PACK
            + lax.shift_right_logical(r_iota, 4),
            1.0, 0.0)
        out_ref[...] = lax.dot_general(
            xs, perm, (((0,), (0,)), ((), ())),
            preferred_element_type=jnp.float32)

    return pl.pallas_call(
        body,
        grid=(_TC_GRID,),
        in_specs=[pl.BlockSpec((_D, _TC_CHUNK), lambda j: (0, j))],
        out_specs=pl.BlockSpec((_SUB, _PACK * _D), lambda j: (j, 0)),
        out_shape=jax.ShapeDtypeStruct((_SLOTS, _PACK * _D), jnp.float32),
    )(table_t)


@functools.lru_cache(maxsize=None)
def _build_sc_kernel():
    info = plsc.get_sparse_core_info()
    nc, ns = info.num_cores, info.num_subcores
    nw = nc * ns
    bpw = _NUM_ANSWERS // nw
    chunk = 128
    nchunks = bpw // chunk
    mesh = plsc.VectorSubcoreMesh(core_axis_name="c", subcore_axis_name="s")

    @functools.partial(
        pl.kernel,
        mesh=mesh,
        compiler_params=pltpu.CompilerParams(needs_layout_passes=False),
        out_type=(
            jax.ShapeDtypeStruct((_NUM_ANSWERS,), jnp.float32),
            jax.ShapeDtypeStruct((_NUM_ANSWERS,), jnp.float32),
        ),
        scratch_types=[
            pltpu.VMEM((bpw,), jnp.int32),
            pltpu.VMEM((bpw,), jnp.int32),
            pltpu.VMEM((bpw,), jnp.int32),
            pltpu.VMEM((bpw,), jnp.int32),
            pltpu.VMEM((bpw,), jnp.int32),
            pltpu.VMEM((bpw,), jnp.int32),
            pltpu.VMEM((chunk, _PACK * _D), jnp.float32),
            pltpu.VMEM((chunk, _PACK * _D), jnp.float32),
            pltpu.VMEM((chunk, _PACK * _D), jnp.float32),
            pltpu.VMEM((chunk, _PACK * _D), jnp.float32),
            pltpu.VMEM((chunk, _PACK * _D), jnp.float32),
            pltpu.VMEM((chunk, _PACK * _D), jnp.float32),
            pltpu.VMEM((bpw,), jnp.float32),
            pltpu.VMEM((bpw,), jnp.float32),
            pltpu.SemaphoreType.DMA,
            pltpu.SemaphoreType.DMA,
        ],
    )
    def sc_kernel(hidx_hbm, widx_hbm, lidx_hbm, table_hbm, win_hbm, lose_hbm,
                  hslot_v, wslot_v, lslot_v, hsub_v, wsub_v, lsub_v,
                  hbuf0, wbuf0, lbuf0, hbuf1, wbuf1, lbuf1,
                  win_v, lose_v, sem0, sem1):
        wid = lax.axis_index("s") * nc + lax.axis_index("c")
        base = wid * bpw
        pltpu.sync_copy(hidx_hbm.at[pl.ds(base, bpw)], hslot_v)
        pltpu.sync_copy(widx_hbm.at[pl.ds(base, bpw)], wslot_v)
        pltpu.sync_copy(lidx_hbm.at[pl.ds(base, bpw)], lslot_v)

        def split_body(g, carry):
            off = g * 16
            for slot_v, sub_v in (
                (hslot_v, hsub_v), (wslot_v, wsub_v), (lslot_v, lsub_v),
            ):
                raw = slot_v[pl.ds(off, 16)]
                sub_v[pl.ds(off, 16)] = (
                    lax.shift_right_logical(raw, _SUB_SHIFT) & (_PACK - 1))
                slot_v[pl.ds(off, 16)] = (
                    lax.shift_left(
                        lax.shift_right_logical(raw, _CHUNK_SHIFT), _SUB_SHIFT)
                    | (raw & (_SUB - 1)))
            return carry

        lax.fori_loop(0, bpw // 16, split_body, 0)

        row_iota = lax.iota(jnp.int32, 16)

        bufs = ((hbuf0, wbuf0, lbuf0), (hbuf1, wbuf1, lbuf1))
        sems = (sem0, sem1)

        def fire(ci, pp):
            coff = ci * chunk
            return [
                pltpu.async_copy(
                    table_hbm.at[slot_v.at[pl.ds(coff, chunk)]],
                    bufs[pp][r], sems[pp])
                for r, slot_v in enumerate((hslot_v, wslot_v, lslot_v))
            ]

        def compute(ci, pp):
            coff = ci * chunk
            hbuf, wbuf, lbuf = bufs[pp]

            def group_body(g, inner):
                goff = g * 16
                rows = goff + row_iota
                hs = hsub_v[pl.ds(coff + goff, 16)]
                ws = wsub_v[pl.ds(coff + goff, 16)]
                ls = lsub_v[pl.ds(coff + goff, 16)]
                wacc = jnp.zeros((16,), jnp.float32)
                lacc = jnp.zeros((16,), jnp.float32)
                for d in range(_D):
                    h = plsc.load_gather(hbuf, [rows, hs + d * _PACK])
                    w = plsc.load_gather(wbuf, [rows, ws + d * _PACK])
                    l = plsc.load_gather(lbuf, [rows, ls + d * _PACK])
                    dw = h - w
                    dl = h - l
                    wacc = wacc + dw * dw
                    lacc = lacc + dl * dl
                win_v[pl.ds(coff + goff, 16)] = wacc
                lose_v[pl.ds(coff + goff, 16)] = lacc
                return inner

            lax.fori_loop(0, chunk // 16, group_body, 0)

        cps = fire(0, 0)
        for ci in range(nchunks):
            pp = ci % 2
            for cp in cps:
                cp.wait()
            nxt = []
            if ci + 1 < nchunks:
                nxt = fire(ci + 1, (ci + 1) % 2)
            compute(ci, pp)
            cps = nxt
        pltpu.sync_copy(win_v, win_hbm.at[pl.ds(base, bpw)])
        pltpu.sync_copy(lose_v, lose_hbm.at[pl.ds(base, bpw)])

    return sc_kernel


def kernel(h_w_l, embedding):
    table_t = embedding.T
    rows_tbl = _tc_repack(table_t)
    hidx = h_w_l[:, 0]
    widx = h_w_l[:, 1]
    lidx = h_w_l[:, 2]
    win, lose = _build_sc_kernel()(hidx, widx, lidx, rows_tbl)
    return (win, lose)


# speedup vs baseline: 6.6338x; 1.0810x over previous
import functools

import jax
import jax.numpy as jnp
from jax import lax
from jax.experimental import pallas as pl
from jax.experimental.pallas import tpu as pltpu
from jax.experimental.pallas import tpu_sc as plsc

_NUM_ANSWERS = 16384
_N = 1000000
_D = 16
_PACK = 16
_NW = 8
_TC_CHUNK = 131072
_TC_GRID = -(-_N // _TC_CHUNK)
_SUB = _TC_CHUNK // _PACK
_SLOTS = _TC_GRID * _SUB
_SUB_SHIFT = _SUB.bit_length() - 1
_CHUNK_SHIFT = _TC_CHUNK.bit_length() - 1


def _tc_repack(table_t):
    def body(in_ref, out_ref):
        x = in_ref[...]
        xs = jnp.concatenate(
            [x[:, s * _SUB:(s + 1) * _SUB] for s in range(_PACK)], axis=0)
        r_iota = lax.broadcasted_iota(jnp.int32, (_PACK * _D, _PACK * _NW), 0)
        c_iota = lax.broadcasted_iota(jnp.int32, (_PACK * _D, _PACK * _NW), 1)
        s_r = lax.shift_right_logical(r_iota, 4)
        d_r = r_iota & (_D - 1)
        tgt = lax.shift_right_logical(d_r, 1) * _PACK + s_r
        e_even = jnp.where((c_iota == tgt) & ((d_r & 1) == 0), 1.0, 0.0)
        e_odd = jnp.where((c_iota == tgt) & ((d_r & 1) == 1), 1.0, 0.0)
        dims = (((0,), (0,)), ((), ()))
        out_e = lax.dot_general(xs, e_even, dims,
                                preferred_element_type=jnp.float32)
        out_o = lax.dot_general(xs, e_odd, dims,
                                preferred_element_type=jnp.float32)
        ue = lax.bitcast_convert_type(
            out_e.astype(jnp.bfloat16), jnp.uint16).astype(jnp.uint32)
        uo = lax.bitcast_convert_type(
            out_o.astype(jnp.bfloat16), jnp.uint16).astype(jnp.uint32)
        word = ue | lax.shift_left(uo, jnp.uint32(16))
        out_ref[...] = lax.bitcast_convert_type(word, jnp.float32)

    return pl.pallas_call(
        body,
        grid=(_TC_GRID,),
        in_specs=[pl.BlockSpec((_D, _TC_CHUNK), lambda j: (0, j))],
        out_specs=pl.BlockSpec((_SUB, _PACK * _NW), lambda j: (j, 0)),
        out_shape=jax.ShapeDtypeStruct((_SLOTS, _PACK * _NW), jnp.float32),
    )(table_t)


@functools.lru_cache(maxsize=None)
def _build_sc_kernel():
    info = plsc.get_sparse_core_info()
    nc, ns = info.num_cores, info.num_subcores
    nw = nc * ns
    bpw = _NUM_ANSWERS // nw
    chunk = 128
    nchunks = bpw // chunk
    mesh = plsc.VectorSubcoreMesh(core_axis_name="c", subcore_axis_name="s")

    @functools.partial(
        pl.kernel,
        mesh=mesh,
        compiler_params=pltpu.CompilerParams(needs_layout_passes=False),
        out_type=(
            jax.ShapeDtypeStruct((_NUM_ANSWERS,), jnp.float32),
            jax.ShapeDtypeStruct((_NUM_ANSWERS,), jnp.float32),
        ),
        scratch_types=[
            pltpu.VMEM((bpw,), jnp.int32),
            pltpu.VMEM((bpw,), jnp.int32),
            pltpu.VMEM((bpw,), jnp.int32),
            pltpu.VMEM((bpw,), jnp.int32),
            pltpu.VMEM((bpw,), jnp.int32),
            pltpu.VMEM((bpw,), jnp.int32),
            pltpu.VMEM((chunk, _PACK * _NW), jnp.float32),
            pltpu.VMEM((chunk, _PACK * _NW), jnp.float32),
            pltpu.VMEM((chunk, _PACK * _NW), jnp.float32),
            pltpu.VMEM((chunk, _PACK * _NW), jnp.float32),
            pltpu.VMEM((chunk, _PACK * _NW), jnp.float32),
            pltpu.VMEM((chunk, _PACK * _NW), jnp.float32),
            pltpu.VMEM((bpw,), jnp.float32),
            pltpu.VMEM((bpw,), jnp.float32),
            pltpu.SemaphoreType.DMA,
            pltpu.SemaphoreType.DMA,
        ],
    )
    def sc_kernel(hidx_hbm, widx_hbm, lidx_hbm, table_hbm, win_hbm, lose_hbm,
                  hslot_v, wslot_v, lslot_v, hsub_v, wsub_v, lsub_v,
                  hbuf0, wbuf0, lbuf0, hbuf1, wbuf1, lbuf1,
                  win_v, lose_v, sem0, sem1):
        wid = lax.axis_index("s") * nc + lax.axis_index("c")
        base = wid * bpw
        pltpu.sync_copy(hidx_hbm.at[pl.ds(base, bpw)], hslot_v)
        pltpu.sync_copy(widx_hbm.at[pl.ds(base, bpw)], wslot_v)
        pltpu.sync_copy(lidx_hbm.at[pl.ds(base, bpw)], lslot_v)

        def split_body(g, carry):
            off = g * 16
            for slot_v, sub_v in (
                (hslot_v, hsub_v), (wslot_v, wsub_v), (lslot_v, lsub_v),
            ):
                raw = slot_v[pl.ds(off, 16)]
                sub_v[pl.ds(off, 16)] = (
                    lax.shift_right_logical(raw, _SUB_SHIFT) & (_PACK - 1))
                slot_v[pl.ds(off, 16)] = (
                    lax.shift_left(
                        lax.shift_right_logical(raw, _CHUNK_SHIFT), _SUB_SHIFT)
                    | (raw & (_SUB - 1)))
            return carry

        lax.fori_loop(0, bpw // 16, split_body, 0)

        row_iota = lax.iota(jnp.int32, 16)

        bufs = ((hbuf0, wbuf0, lbuf0), (hbuf1, wbuf1, lbuf1))
        sems = (sem0, sem1)

        def fire(ci, pp):
            coff = ci * chunk
            return [
                pltpu.async_copy(
                    table_hbm.at[slot_v.at[pl.ds(coff, chunk)]],
                    bufs[pp][r], sems[pp])
                for r, slot_v in enumerate((hslot_v, wslot_v, lslot_v))
            ]

        def compute(ci, pp):
            coff = ci * chunk
            hbuf, wbuf, lbuf = bufs[pp]

            def group_body(g, inner):
                goff = g * 16
                rows = goff + row_iota
                hs = hsub_v[pl.ds(coff + goff, 16)]
                ws = wsub_v[pl.ds(coff + goff, 16)]
                ls = lsub_v[pl.ds(coff + goff, 16)]
                wacc = jnp.zeros((16,), jnp.float32)
                lacc = jnp.zeros((16,), jnp.float32)
                himask = jnp.full((16,), -65536, jnp.int32)

                def halves(word_f32):
                    bits = plsc.bitcast(word_f32, jnp.int32)
                    lo = plsc.bitcast(lax.shift_left(bits, 16), jnp.float32)
                    hi = plsc.bitcast(bits & himask, jnp.float32)
                    return lo, hi

                for wi in range(_NW):
                    h2 = halves(plsc.load_gather(hbuf, [rows, hs + wi * _PACK]))
                    w2 = halves(plsc.load_gather(wbuf, [rows, ws + wi * _PACK]))
                    l2 = halves(plsc.load_gather(lbuf, [rows, ls + wi * _PACK]))
                    for h, w, l in ((h2[0], w2[0], l2[0]),
                                    (h2[1], w2[1], l2[1])):
                        dw = h - w
                        dl = h - l
                        wacc = wacc + dw * dw
                        lacc = lacc + dl * dl
                win_v[pl.ds(coff + goff, 16)] = wacc
                lose_v[pl.ds(coff + goff, 16)] = lacc
                return inner

            lax.fori_loop(0, chunk // 16, group_body, 0)

        cps = fire(0, 0)
        for ci in range(nchunks):
            pp = ci % 2
            for cp in cps:
                cp.wait()
            nxt = []
            if ci + 1 < nchunks:
                nxt = fire(ci + 1, (ci + 1) % 2)
            compute(ci, pp)
            cps = nxt
        pltpu.sync_copy(win_v, win_hbm.at[pl.ds(base, bpw)])
        pltpu.sync_copy(lose_v, lose_hbm.at[pl.ds(base, bpw)])

    return sc_kernel


def kernel(h_w_l, embedding):
    table_t = embedding.T
    rows_tbl = _tc_repack(table_t)
    hidx = h_w_l[:, 0]
    widx = h_w_l[:, 1]
    lidx = h_w_l[:, 2]
    win, lose = _build_sc_kernel()(hidx, widx, lidx, rows_tbl)
    return (win, lose)


# speedup vs baseline: 6.8196x; 1.0280x over previous
import functools

import jax
import jax.numpy as jnp
from jax import lax
from jax.experimental import pallas as pl
from jax.experimental.pallas import tpu as pltpu
from jax.experimental.pallas import tpu_sc as plsc

_NUM_ANSWERS = 16384
_N = 1000000
_D = 16
_PACK = 16
_NW = 8
_TC_CHUNK = 262144
_TC_GRID = -(-_N // _TC_CHUNK)
_SUB = _TC_CHUNK // _PACK
_SLOTS = _TC_GRID * _SUB
_SUB_SHIFT = _SUB.bit_length() - 1
_CHUNK_SHIFT = _TC_CHUNK.bit_length() - 1


def _tc_repack(table_t):
    def body(in_ref, out_ref):
        x = in_ref[...]
        xs = jnp.concatenate(
            [x[:, s * _SUB:(s + 1) * _SUB] for s in range(_PACK)], axis=0)
        r_iota = lax.broadcasted_iota(jnp.int32, (_PACK * _D, _PACK * _NW), 0)
        c_iota = lax.broadcasted_iota(jnp.int32, (_PACK * _D, _PACK * _NW), 1)
        s_r = lax.shift_right_logical(r_iota, 4)
        d_r = r_iota & (_D - 1)
        tgt = lax.shift_right_logical(d_r, 1) * _PACK + s_r
        e_even = jnp.where((c_iota == tgt) & ((d_r & 1) == 0), 1.0, 0.0)
        e_odd = jnp.where((c_iota == tgt) & ((d_r & 1) == 1), 1.0, 0.0)
        dims = (((0,), (0,)), ((), ()))
        out_e = lax.dot_general(xs, e_even, dims,
                                preferred_element_type=jnp.float32)
        out_o = lax.dot_general(xs, e_odd, dims,
                                preferred_element_type=jnp.float32)
        ue = lax.bitcast_convert_type(
            out_e.astype(jnp.bfloat16), jnp.uint16).astype(jnp.uint32)
        uo = lax.bitcast_convert_type(
            out_o.astype(jnp.bfloat16), jnp.uint16).astype(jnp.uint32)
        word = ue | lax.shift_left(uo, jnp.uint32(16))
        out_ref[...] = lax.bitcast_convert_type(word, jnp.float32)

    return pl.pallas_call(
        body,
        grid=(_TC_GRID,),
        in_specs=[pl.BlockSpec((_D, _TC_CHUNK), lambda j: (0, j))],
        out_specs=pl.BlockSpec((_SUB, _PACK * _NW), lambda j: (j, 0)),
        out_shape=jax.ShapeDtypeStruct((_SLOTS, _PACK * _NW), jnp.float32),
    )(table_t)


@functools.lru_cache(maxsize=None)
def _build_sc_kernel():
    info = plsc.get_sparse_core_info()
    nc, ns = info.num_cores, info.num_subcores
    nw = nc * ns
    bpw = _NUM_ANSWERS // nw
    chunk = 128
    nchunks = bpw // chunk
    mesh = plsc.VectorSubcoreMesh(core_axis_name="c", subcore_axis_name="s")

    @functools.partial(
        pl.kernel,
        mesh=mesh,
        compiler_params=pltpu.CompilerParams(needs_layout_passes=False),
        out_type=(
            jax.ShapeDtypeStruct((_NUM_ANSWERS,), jnp.float32),
            jax.ShapeDtypeStruct((_NUM_ANSWERS,), jnp.float32),
        ),
        scratch_types=[
            pltpu.VMEM((bpw,), jnp.int32),
            pltpu.VMEM((bpw,), jnp.int32),
            pltpu.VMEM((bpw,), jnp.int32),
            pltpu.VMEM((bpw,), jnp.int32),
            pltpu.VMEM((bpw,), jnp.int32),
            pltpu.VMEM((bpw,), jnp.int32),
            pltpu.VMEM((chunk, _PACK * _NW), jnp.float32),
            pltpu.VMEM((chunk, _PACK * _NW), jnp.float32),
            pltpu.VMEM((chunk, _PACK * _NW), jnp.float32),
            pltpu.VMEM((chunk, _PACK * _NW), jnp.float32),
            pltpu.VMEM((chunk, _PACK * _NW), jnp.float32),
            pltpu.VMEM((chunk, _PACK * _NW), jnp.float32),
            pltpu.VMEM((bpw,), jnp.float32),
            pltpu.VMEM((bpw,), jnp.float32),
            pltpu.SemaphoreType.DMA,
            pltpu.SemaphoreType.DMA,
        ],
    )
    def sc_kernel(hidx_hbm, widx_hbm, lidx_hbm, table_hbm, win_hbm, lose_hbm,
                  hslot_v, wslot_v, lslot_v, hsub_v, wsub_v, lsub_v,
                  hbuf0, wbuf0, lbuf0, hbuf1, wbuf1, lbuf1,
                  win_v, lose_v, sem0, sem1):
        wid = lax.axis_index("s") * nc + lax.axis_index("c")
        base = wid * bpw
        pltpu.sync_copy(hidx_hbm.at[pl.ds(base, bpw)], hslot_v)
        pltpu.sync_copy(widx_hbm.at[pl.ds(base, bpw)], wslot_v)
        pltpu.sync_copy(lidx_hbm.at[pl.ds(base, bpw)], lslot_v)

        def split_body(g, carry):
            off = g * 16
            for slot_v, sub_v in (
                (hslot_v, hsub_v), (wslot_v, wsub_v), (lslot_v, lsub_v),
            ):
                raw = slot_v[pl.ds(off, 16)]
                sub_v[pl.ds(off, 16)] = (
                    lax.shift_right_logical(raw, _SUB_SHIFT) & (_PACK - 1))
                slot_v[pl.ds(off, 16)] = (
                    lax.shift_left(
                        lax.shift_right_logical(raw, _CHUNK_SHIFT), _SUB_SHIFT)
                    | (raw & (_SUB - 1)))
            return carry

        lax.fori_loop(0, bpw // 16, split_body, 0)

        row_iota = lax.iota(jnp.int32, 16)

        bufs = ((hbuf0, wbuf0, lbuf0), (hbuf1, wbuf1, lbuf1))
        sems = (sem0, sem1)

        def fire(ci, pp):
            coff = ci * chunk
            return [
                pltpu.async_copy(
                    table_hbm.at[slot_v.at[pl.ds(coff, chunk)]],
                    bufs[pp][r], sems[pp])
                for r, slot_v in enumerate((hslot_v, wslot_v, lslot_v))
            ]

        def compute(ci, pp):
            coff = ci * chunk
            hbuf, wbuf, lbuf = bufs[pp]

            def group_body(g, inner):
                goff = g * 16
                rows = goff + row_iota
                hs = hsub_v[pl.ds(coff + goff, 16)]
                ws = wsub_v[pl.ds(coff + goff, 16)]
                ls = lsub_v[pl.ds(coff + goff, 16)]
                wacc = jnp.zeros((16,), jnp.float32)
                lacc = jnp.zeros((16,), jnp.float32)
                himask = jnp.full((16,), -65536, jnp.int32)

                def halves(word_f32):
                    bits = plsc.bitcast(word_f32, jnp.int32)
                    lo = plsc.bitcast(lax.shift_left(bits, 16), jnp.float32)
                    hi = plsc.bitcast(bits & himask, jnp.float32)
                    return lo, hi

                for wi in range(_NW):
                    h2 = halves(plsc.load_gather(hbuf, [rows, hs + wi * _PACK]))
                    w2 = halves(plsc.load_gather(wbuf, [rows, ws + wi * _PACK]))
                    l2 = halves(plsc.load_gather(lbuf, [rows, ls + wi * _PACK]))
                    for h, w, l in ((h2[0], w2[0], l2[0]),
                                    (h2[1], w2[1], l2[1])):
                        dw = h - w
                        dl = h - l
                        wacc = wacc + dw * dw
                        lacc = lacc + dl * dl
                win_v[pl.ds(coff + goff, 16)] = wacc
                lose_v[pl.ds(coff + goff, 16)] = lacc
                return inner

            lax.fori_loop(0, chunk // 16, group_body, 0)

        cps = fire(0, 0)
        for ci in range(nchunks):
            pp = ci % 2
            for cp in cps:
                cp.wait()
            nxt = []
            if ci + 1 < nchunks:
                nxt = fire(ci + 1, (ci + 1) % 2)
            compute(ci, pp)
            cps = nxt
        pltpu.sync_copy(win_v, win_hbm.at[pl.ds(base, bpw)])
        pltpu.sync_copy(lose_v, lose_hbm.at[pl.ds(base, bpw)])

    return sc_kernel


def kernel(h_w_l, embedding):
    table_t = embedding.T
    rows_tbl = _tc_repack(table_t)
    hidx = h_w_l[:, 0]
    widx = h_w_l[:, 1]
    lidx = h_w_l[:, 2]
    win, lose = _build_sc_kernel()(hidx, widx, lidx, rows_tbl)
    return (win, lose)


# speedup vs baseline: 6.8213x; 1.0002x over previous
import functools

import jax
import jax.numpy as jnp
from jax import lax
from jax.experimental import pallas as pl
from jax.experimental.pallas import tpu as pltpu
from jax.experimental.pallas import tpu_sc as plsc

_NUM_ANSWERS = 16384
_N = 1000000
_D = 16
_PACK = 16
_NW = 8
_TC_CHUNK = 262144
_TC_GRID = -(-_N // _TC_CHUNK)
_SUB = _TC_CHUNK // _PACK
_SLOTS = _TC_GRID * _SUB
_SUB_SHIFT = _SUB.bit_length() - 1
_CHUNK_SHIFT = _TC_CHUNK.bit_length() - 1


def _tc_repack(table_t):
    def body(in_ref, out_ref):
        x = in_ref[...]
        xs = jnp.concatenate(
            [x[:, s * _SUB:(s + 1) * _SUB] for s in range(_PACK)], axis=0)
        r_iota = lax.broadcasted_iota(jnp.int32, (_PACK * _D, _PACK * _NW), 0)
        c_iota = lax.broadcasted_iota(jnp.int32, (_PACK * _D, _PACK * _NW), 1)
        s_r = lax.shift_right_logical(r_iota, 4)
        d_r = r_iota & (_D - 1)
        tgt = lax.shift_right_logical(d_r, 1) * _PACK + s_r
        e_even = jnp.where((c_iota == tgt) & ((d_r & 1) == 0), 1.0, 0.0)
        e_odd = jnp.where((c_iota == tgt) & ((d_r & 1) == 1), 1.0, 0.0)
        dims = (((0,), (0,)), ((), ()))
        out_e = lax.dot_general(xs, e_even, dims,
                                preferred_element_type=jnp.float32)
        out_o = lax.dot_general(xs, e_odd, dims,
                                preferred_element_type=jnp.float32)
        ue = lax.bitcast_convert_type(
            out_e.astype(jnp.bfloat16), jnp.uint16).astype(jnp.uint32)
        uo = lax.bitcast_convert_type(
            out_o.astype(jnp.bfloat16), jnp.uint16).astype(jnp.uint32)
        word = ue | lax.shift_left(uo, jnp.uint32(16))
        out_ref[...] = lax.bitcast_convert_type(word, jnp.float32)

    return pl.pallas_call(
        body,
        grid=(_TC_GRID,),
        in_specs=[pl.BlockSpec((_D, _TC_CHUNK), lambda j: (0, j))],
        out_specs=pl.BlockSpec((_SUB, _PACK * _NW), lambda j: (j, 0)),
        out_shape=jax.ShapeDtypeStruct((_SLOTS, _PACK * _NW), jnp.float32),
    )(table_t)


@functools.lru_cache(maxsize=None)
def _build_sc_kernel():
    info = plsc.get_sparse_core_info()
    nc, ns = info.num_cores, info.num_subcores
    nw = nc * ns
    bpw = _NUM_ANSWERS // nw
    chunk = 128
    nchunks = bpw // chunk
    mesh = plsc.VectorSubcoreMesh(core_axis_name="c", subcore_axis_name="s")

    @functools.partial(
        pl.kernel,
        mesh=mesh,
        compiler_params=pltpu.CompilerParams(needs_layout_passes=False),
        out_type=(
            jax.ShapeDtypeStruct((_NUM_ANSWERS,), jnp.float32),
            jax.ShapeDtypeStruct((_NUM_ANSWERS,), jnp.float32),
        ),
        scratch_types=[
            pltpu.VMEM((bpw,), jnp.int32),
            pltpu.VMEM((bpw,), jnp.int32),
            pltpu.VMEM((bpw,), jnp.int32),
            pltpu.VMEM((bpw,), jnp.int32),
            pltpu.VMEM((bpw,), jnp.int32),
            pltpu.VMEM((bpw,), jnp.int32),
            pltpu.VMEM((chunk, _PACK * _NW), jnp.float32),
            pltpu.VMEM((chunk, _PACK * _NW), jnp.float32),
            pltpu.VMEM((chunk, _PACK * _NW), jnp.float32),
            pltpu.VMEM((chunk, _PACK * _NW), jnp.float32),
            pltpu.VMEM((chunk, _PACK * _NW), jnp.float32),
            pltpu.VMEM((chunk, _PACK * _NW), jnp.float32),
            pltpu.VMEM((bpw,), jnp.float32),
            pltpu.VMEM((bpw,), jnp.float32),
            pltpu.SemaphoreType.DMA,
            pltpu.SemaphoreType.DMA,
        ],
    )
    def sc_kernel(idx_hbm, table_hbm, win_hbm, lose_hbm,
                  hslot_v, wslot_v, lslot_v, hsub_v, wsub_v, lsub_v,
                  hbuf0, wbuf0, lbuf0, hbuf1, wbuf1, lbuf1,
                  win_v, lose_v, sem0, sem1):
        wid = lax.axis_index("s") * nc + lax.axis_index("c")
        base = wid * bpw
        pltpu.sync_copy(idx_hbm.at[pl.ds(base, bpw)], hslot_v)
        pltpu.sync_copy(idx_hbm.at[pl.ds(_NUM_ANSWERS + base, bpw)], wslot_v)
        pltpu.sync_copy(idx_hbm.at[pl.ds(2 * _NUM_ANSWERS + base, bpw)],
                        lslot_v)

        def split_body(g, carry):
            off = g * 16
            for slot_v, sub_v in (
                (hslot_v, hsub_v), (wslot_v, wsub_v), (lslot_v, lsub_v),
            ):
                raw = slot_v[pl.ds(off, 16)]
                sub_v[pl.ds(off, 16)] = (
                    lax.shift_right_logical(raw, _SUB_SHIFT) & (_PACK - 1))
                slot_v[pl.ds(off, 16)] = (
                    lax.shift_left(
                        lax.shift_right_logical(raw, _CHUNK_SHIFT), _SUB_SHIFT)
                    | (raw & (_SUB - 1)))
            return carry

        lax.fori_loop(0, bpw // 16, split_body, 0)

        row_iota = lax.iota(jnp.int32, 16)

        bufs = ((hbuf0, wbuf0, lbuf0), (hbuf1, wbuf1, lbuf1))
        sems = (sem0, sem1)

        def fire(ci, pp):
            coff = ci * chunk
            return [
                pltpu.async_copy(
                    table_hbm.at[slot_v.at[pl.ds(coff, chunk)]],
                    bufs[pp][r], sems[pp])
                for r, slot_v in enumerate((hslot_v, wslot_v, lslot_v))
            ]

        def compute(ci, pp):
            coff = ci * chunk
            hbuf, wbuf, lbuf = bufs[pp]

            def group_body(g, inner):
                goff = g * 16
                rows = goff + row_iota
                hs = hsub_v[pl.ds(coff + goff, 16)]
                ws = wsub_v[pl.ds(coff + goff, 16)]
                ls = lsub_v[pl.ds(coff + goff, 16)]
                wacc = jnp.zeros((16,), jnp.float32)
                lacc = jnp.zeros((16,), jnp.float32)
                himask = jnp.full((16,), -65536, jnp.int32)

                def halves(word_f32):
                    bits = plsc.bitcast(word_f32, jnp.int32)
                    lo = plsc.bitcast(lax.shift_left(bits, 16), jnp.float32)
                    hi = plsc.bitcast(bits & himask, jnp.float32)
                    return lo, hi

                for wi in range(_NW):
                    h2 = halves(plsc.load_gather(hbuf, [rows, hs + wi * _PACK]))
                    w2 = halves(plsc.load_gather(wbuf, [rows, ws + wi * _PACK]))
                    l2 = halves(plsc.load_gather(lbuf, [rows, ls + wi * _PACK]))
                    for h, w, l in ((h2[0], w2[0], l2[0]),
                                    (h2[1], w2[1], l2[1])):
                        dw = h - w
                        dl = h - l
                        wacc = wacc + dw * dw
                        lacc = lacc + dl * dl
                win_v[pl.ds(coff + goff, 16)] = wacc
                lose_v[pl.ds(coff + goff, 16)] = lacc
                return inner

            lax.fori_loop(0, chunk // 16, group_body, 0)

        cps = fire(0, 0)
        for ci in range(nchunks):
            pp = ci % 2
            for cp in cps:
                cp.wait()
            nxt = []
            if ci + 1 < nchunks:
                nxt = fire(ci + 1, (ci + 1) % 2)
            compute(ci, pp)
            cps = nxt
        pltpu.sync_copy(win_v, win_hbm.at[pl.ds(base, bpw)])
        pltpu.sync_copy(lose_v, lose_hbm.at[pl.ds(base, bpw)])

    return sc_kernel


def kernel(h_w_l, embedding):
    table_t = embedding.T
    rows_tbl = _tc_repack(table_t)
    win, lose = _build_sc_kernel()(h_w_l.T.reshape(-1), rows_tbl)
    return (win, lose)
